# Initial kernel scaffold; baseline (speedup 1.0000x reference)
import functools

import jax
import jax.numpy as jnp
from jax import lax
from jax.experimental import pallas as pl
from jax.experimental.pallas import tpu as pltpu
from jax.experimental.pallas import tpu_sc as plsc

N = 10000
E = 160000
IN_DIM = 256
HID = 256
OUT = 64

NP = 10240
EP = 163840
CH = 128
NTILES = 16
RPT = NP // NTILES

_mesh = plsc.VectorSubcoreMesh(core_axis_name="c", subcore_axis_name="s")


@functools.partial(
    pl.kernel,
    out_type=jax.ShapeDtypeStruct((2, NP, 16), jnp.float32),
    mesh=_mesh,
    scratch_types=[
        pltpu.VMEM((CH,), jnp.int32),
        pltpu.VMEM((CH, 16), jnp.float32),
        pltpu.VMEM_SHARED((NP, 16), jnp.float32),
    ],
)
def _k1_deg(dst_hbm, zeros_hbm, ones_hbm, out_hbm, idx_v, ones_v, acc):
    cid = lax.axis_index("c")
    sid = lax.axis_index("s")
    pltpu.sync_copy(
        zeros_hbm.at[pl.ds(sid * RPT, RPT), pl.ds(0, 16)],
        acc.at[pl.ds(sid * RPT, RPT)],
    )
    pltpu.sync_copy(ones_hbm, ones_v)
    plsc.subcore_barrier()
    ept = EP // 32
    base = (cid * NTILES + sid) * ept

    def body(i, carry):
        pltpu.sync_copy(dst_hbm.at[pl.ds(base + i * CH, CH)], idx_v)
        pltpu.sync_copy(ones_v, acc.at[idx_v], add=True)
        return carry

    lax.fori_loop(0, ept // CH, body, 0)
    plsc.subcore_barrier()
    pltpu.sync_copy(
        acc.at[pl.ds(sid * RPT, RPT)],
        out_hbm.at[cid, pl.ds(sid * RPT, RPT)],
    )


@functools.partial(
    pl.kernel,
    out_type=jax.ShapeDtypeStruct((2, NP, 128), jnp.float32),
    mesh=_mesh,
    scratch_types=[
        pltpu.VMEM((CH,), jnp.int32),
        pltpu.VMEM((CH,), jnp.int32),
        pltpu.VMEM((CH, 128), jnp.float32),
        pltpu.VMEM_SHARED((NP, 128), jnp.float32),
        pltpu.SemaphoreType.DMA,
    ],
)
def _k3_agg1(src_hbm, dst_hbm, ha_hbm, hb_hbm, zeros_hbm, out_hbm,
             src_v, dst_v, rows_v, acc, sem):
    cid = lax.axis_index("c")
    sid = lax.axis_index("s")
    pltpu.sync_copy(
        zeros_hbm.at[pl.ds(sid * RPT, RPT)], acc.at[pl.ds(sid * RPT, RPT)]
    )
    plsc.subcore_barrier()
    ept = EP // NTILES
    base = sid * ept

    def make_body(tbl):
        def body(i, carry):
            pltpu.sync_copy(src_hbm.at[pl.ds(base + i * CH, CH)], src_v)
            pltpu.sync_copy(dst_hbm.at[pl.ds(base + i * CH, CH)], dst_v)
            pltpu.async_copy(tbl.at[src_v], rows_v, sem).wait()
            pltpu.sync_copy(rows_v, acc.at[dst_v], add=True)
            return carry
        return body

    @pl.when(cid == 0)
    def _():
        lax.fori_loop(0, ept // CH, make_body(ha_hbm), 0)

    @pl.when(cid == 1)
    def _():
        lax.fori_loop(0, ept // CH, make_body(hb_hbm), 0)

    plsc.subcore_barrier()
    pltpu.sync_copy(
        acc.at[pl.ds(sid * RPT, RPT)],
        out_hbm.at[cid, pl.ds(sid * RPT, RPT)],
    )


@functools.partial(
    pl.kernel,
    out_type=jax.ShapeDtypeStruct((2, NP, OUT), jnp.float32),
    mesh=_mesh,
    scratch_types=[
        pltpu.VMEM((CH,), jnp.int32),
        pltpu.VMEM((CH,), jnp.int32),
        pltpu.VMEM((CH, OUT), jnp.float32),
        pltpu.VMEM_SHARED((NP, OUT), jnp.float32),
        pltpu.SemaphoreType.DMA,
    ],
)
def _k5_agg2(src_hbm, dst_hbm, h2p_hbm, zeros_hbm, out_hbm,
             src_v, dst_v, rows_v, acc, sem):
    cid = lax.axis_index("c")
    sid = lax.axis_index("s")
    pltpu.sync_copy(
        zeros_hbm.at[pl.ds(sid * RPT, RPT), pl.ds(0, OUT)],
        acc.at[pl.ds(sid * RPT, RPT)],
    )
    plsc.subcore_barrier()
    ept = EP // 32
    base = cid * (EP // 2) + sid * ept

    def body(i, carry):
        pltpu.sync_copy(src_hbm.at[pl.ds(base + i * CH, CH)], src_v)
        pltpu.sync_copy(dst_hbm.at[pl.ds(base + i * CH, CH)], dst_v)
        pltpu.async_copy(h2p_hbm.at[src_v], rows_v, sem).wait()
        pltpu.sync_copy(rows_v, acc.at[dst_v], add=True)
        return carry

    lax.fori_loop(0, ept // CH, body, 0)
    plsc.subcore_barrier()
    pltpu.sync_copy(
        acc.at[pl.ds(sid * RPT, RPT)],
        out_hbm.at[cid, pl.ds(sid * RPT, RPT)],
    )


BLK = 256
GRID = NP // BLK


def _k2_body(deg0_ref, deg1_ref, x_ref, w1_ref, dinv_ref, ha_ref, hb_ref):
    deg = deg0_ref[:, 0] + deg1_ref[:, 0] + 1.0
    dinv = lax.rsqrt(deg)[:, None]
    h = jnp.dot(x_ref[...], w1_ref[...], preferred_element_type=jnp.float32)
    dinv_ref[...] = jnp.broadcast_to(dinv, (BLK, 128))
    ha_ref[...] = h[:, :128] * dinv
    hb_ref[...] = h[:, 128:] * dinv


_k2 = pl.pallas_call(
    _k2_body,
    grid=(GRID,),
    in_specs=[
        pl.BlockSpec((BLK, 16), lambda i: (i, 0)),
        pl.BlockSpec((BLK, 16), lambda i: (i, 0)),
        pl.BlockSpec((BLK, IN_DIM), lambda i: (i, 0)),
        pl.BlockSpec((IN_DIM, HID), lambda i: (0, 0)),
    ],
    out_specs=[
        pl.BlockSpec((BLK, 128), lambda i: (i, 0)),
        pl.BlockSpec((BLK, 128), lambda i: (i, 0)),
        pl.BlockSpec((BLK, 128), lambda i: (i, 0)),
    ],
    out_shape=[
        jax.ShapeDtypeStruct((NP, 128), jnp.float32),
        jax.ShapeDtypeStruct((NP, 128), jnp.float32),
        jax.ShapeDtypeStruct((NP, 128), jnp.float32),
    ],
)


def _k4_body(dinv_ref, a1a_ref, a1b_ref, ha_ref, hb_ref, b1_ref, w2_ref,
             h2p_ref):
    dv = dinv_ref[...]
    h1a = jnp.maximum(dv * (a1a_ref[...] + ha_ref[...]) + b1_ref[0, :128], 0.0)
    h1b = jnp.maximum(dv * (a1b_ref[...] + hb_ref[...]) + b1_ref[0, 128:], 0.0)
    h2 = jnp.dot(h1a, w2_ref[:128], preferred_element_type=jnp.float32)
    h2 = h2 + jnp.dot(h1b, w2_ref[128:], preferred_element_type=jnp.float32)
    h2p_ref[...] = h2 * dv[:, :OUT]


_k4 = pl.pallas_call(
    _k4_body,
    grid=(GRID,),
    in_specs=[
        pl.BlockSpec((BLK, 128), lambda i: (i, 0)),
        pl.BlockSpec((BLK, 128), lambda i: (i, 0)),
        pl.BlockSpec((BLK, 128), lambda i: (i, 0)),
        pl.BlockSpec((BLK, 128), lambda i: (i, 0)),
        pl.BlockSpec((BLK, 128), lambda i: (i, 0)),
        pl.BlockSpec((1, HID), lambda i: (0, 0)),
        pl.BlockSpec((HID, OUT), lambda i: (0, 0)),
    ],
    out_specs=pl.BlockSpec((BLK, OUT), lambda i: (i, 0)),
    out_shape=jax.ShapeDtypeStruct((NP, OUT), jnp.float32),
)


def _k6_body(dinv_ref, p0_ref, p1_ref, h2p_ref, b2_ref, out_ref):
    out_ref[...] = (
        dinv_ref[:, :OUT] * (p0_ref[...] + p1_ref[...] + h2p_ref[...])
        + b2_ref[0, :]
    )


_k6 = pl.pallas_call(
    _k6_body,
    grid=(GRID,),
    in_specs=[
        pl.BlockSpec((BLK, 128), lambda i: (i, 0)),
        pl.BlockSpec((BLK, OUT), lambda i: (i, 0)),
        pl.BlockSpec((BLK, OUT), lambda i: (i, 0)),
        pl.BlockSpec((BLK, OUT), lambda i: (i, 0)),
        pl.BlockSpec((1, OUT), lambda i: (0, 0)),
    ],
    out_specs=pl.BlockSpec((BLK, OUT), lambda i: (i, 0)),
    out_shape=jax.ShapeDtypeStruct((NP, OUT), jnp.float32),
)


@jax.jit
def kernel(x, edge_index, W1, b1, W2, b2):
    ei = edge_index.astype(jnp.int32)
    pad = jnp.full((EP - E,), N, dtype=jnp.int32)
    src = jnp.concatenate([ei[0], pad])
    dst = jnp.concatenate([ei[1], pad])
    xp = jnp.zeros((NP, IN_DIM), jnp.float32).at[:N].set(x)
    zeros = jnp.zeros((NP, 128), jnp.float32)
    ones16 = jnp.ones((CH, 16), jnp.float32)

    degp = _k1_deg(dst, zeros, ones16)
    dinv, ha, hb = _k2(degp[0], degp[1], xp, W1)
    a1 = _k3_agg1(src, dst, ha, hb, zeros)
    h2p = _k4(dinv, a1[0], a1[1], ha, hb, b1.reshape(1, HID), W2)
    a2 = _k5_agg2(src, dst, h2p, zeros)
    out = _k6(dinv, a2[0], a2[1], h2p, b2.reshape(1, OUT))
    return out[:N]


# speedup vs baseline: 5.9276x; 5.9276x over previous
import functools

import jax
import jax.numpy as jnp
from jax import lax
from jax.experimental import pallas as pl
from jax.experimental.pallas import tpu as pltpu
from jax.experimental.pallas import tpu_sc as plsc

N = 10000
E = 160000
IN_DIM = 256
HID = 256
OUT = 64

NP = 10240
EP = 163840
CH = 128
NTILES = 16
RPT = NP // NTILES

_mesh = plsc.VectorSubcoreMesh(core_axis_name="c", subcore_axis_name="s")


@functools.partial(
    pl.kernel,
    out_type=jax.ShapeDtypeStruct((2, NP, 128), jnp.float32),
    mesh=_mesh,
    scratch_types=[
        pltpu.VMEM((CH,), jnp.int32),
        pltpu.VMEM((CH, 128), jnp.float32),
        pltpu.VMEM_SHARED((NP, 128), jnp.float32),
    ],
)
def _k1_deg(dst_hbm, zeros_hbm, ones_hbm, out_hbm, idx_v, ones_v, acc):
    cid = lax.axis_index("c")
    sid = lax.axis_index("s")
    pltpu.sync_copy(
        zeros_hbm.at[pl.ds(sid * RPT, RPT)],
        acc.at[pl.ds(sid * RPT, RPT)],
    )
    pltpu.sync_copy(ones_hbm, ones_v)
    plsc.subcore_barrier()
    ept = EP // 32
    base = (cid * NTILES + sid) * ept

    def body(i, carry):
        pltpu.sync_copy(dst_hbm.at[pl.ds(base + i * CH, CH)], idx_v)
        pltpu.sync_copy(ones_v, acc.at[idx_v], add=True)
        return carry

    lax.fori_loop(0, ept // CH, body, 0)
    plsc.subcore_barrier()
    pltpu.sync_copy(
        acc.at[pl.ds(sid * RPT, RPT)],
        out_hbm.at[cid, pl.ds(sid * RPT, RPT)],
    )


@functools.partial(
    pl.kernel,
    out_type=jax.ShapeDtypeStruct((2, NP, 128), jnp.float32),
    mesh=_mesh,
    scratch_types=[
        pltpu.VMEM((CH,), jnp.int32),
        pltpu.VMEM((CH,), jnp.int32),
        pltpu.VMEM((CH, 128), jnp.float32),
        pltpu.VMEM_SHARED((NP, 128), jnp.float32),
        pltpu.SemaphoreType.DMA,
    ],
)
def _k3_agg1(src_hbm, dst_hbm, ha_hbm, hb_hbm, zeros_hbm, out_hbm,
             src_v, dst_v, rows_v, acc, sem):
    cid = lax.axis_index("c")
    sid = lax.axis_index("s")
    pltpu.sync_copy(
        zeros_hbm.at[pl.ds(sid * RPT, RPT)], acc.at[pl.ds(sid * RPT, RPT)]
    )
    plsc.subcore_barrier()
    ept = EP // NTILES
    base = sid * ept

    def make_body(tbl):
        def body(i, carry):
            pltpu.sync_copy(src_hbm.at[pl.ds(base + i * CH, CH)], src_v)
            pltpu.sync_copy(dst_hbm.at[pl.ds(base + i * CH, CH)], dst_v)
            pltpu.async_copy(tbl.at[src_v], rows_v, sem).wait()
            pltpu.sync_copy(rows_v, acc.at[dst_v], add=True)
            return carry
        return body

    @pl.when(cid == 0)
    def _():
        lax.fori_loop(0, ept // CH, make_body(ha_hbm), 0)

    @pl.when(cid == 1)
    def _():
        lax.fori_loop(0, ept // CH, make_body(hb_hbm), 0)

    plsc.subcore_barrier()
    pltpu.sync_copy(
        acc.at[pl.ds(sid * RPT, RPT)],
        out_hbm.at[cid, pl.ds(sid * RPT, RPT)],
    )


@functools.partial(
    pl.kernel,
    out_type=jax.ShapeDtypeStruct((2, NP, 128), jnp.float32),
    mesh=_mesh,
    scratch_types=[
        pltpu.VMEM((CH,), jnp.int32),
        pltpu.VMEM((CH,), jnp.int32),
        pltpu.VMEM((CH, 128), jnp.float32),
        pltpu.VMEM_SHARED((NP, 128), jnp.float32),
        pltpu.SemaphoreType.DMA,
    ],
)
def _k5_agg2(src_hbm, dst_hbm, h2p_hbm, zeros_hbm, out_hbm,
             src_v, dst_v, rows_v, acc, sem):
    cid = lax.axis_index("c")
    sid = lax.axis_index("s")
    pltpu.sync_copy(
        zeros_hbm.at[pl.ds(sid * RPT, RPT)],
        acc.at[pl.ds(sid * RPT, RPT)],
    )
    plsc.subcore_barrier()
    ept = EP // 32
    base = cid * (EP // 2) + sid * ept

    def body(i, carry):
        pltpu.sync_copy(src_hbm.at[pl.ds(base + i * CH, CH)], src_v)
        pltpu.sync_copy(dst_hbm.at[pl.ds(base + i * CH, CH)], dst_v)
        pltpu.async_copy(h2p_hbm.at[src_v], rows_v, sem).wait()
        pltpu.sync_copy(rows_v, acc.at[dst_v], add=True)
        return carry

    lax.fori_loop(0, ept // CH, body, 0)
    plsc.subcore_barrier()
    pltpu.sync_copy(
        acc.at[pl.ds(sid * RPT, RPT)],
        out_hbm.at[cid, pl.ds(sid * RPT, RPT)],
    )


BLK = 256
GRID = NP // BLK


def _k2_body(deg0_ref, deg1_ref, x_ref, w1_ref, dinv_ref, ha_ref, hb_ref):
    deg = deg0_ref[:, 0] + deg1_ref[:, 0] + 1.0
    dinv = lax.rsqrt(deg)[:, None]
    h = jnp.dot(x_ref[...], w1_ref[...], preferred_element_type=jnp.float32)
    dinv_ref[...] = jnp.broadcast_to(dinv, (BLK, 128))
    ha_ref[...] = h[:, :128] * dinv
    hb_ref[...] = h[:, 128:] * dinv


_k2 = pl.pallas_call(
    _k2_body,
    grid=(GRID,),
    in_specs=[
        pl.BlockSpec((BLK, 128), lambda i: (i, 0)),
        pl.BlockSpec((BLK, 128), lambda i: (i, 0)),
        pl.BlockSpec((BLK, IN_DIM), lambda i: (i, 0)),
        pl.BlockSpec((IN_DIM, HID), lambda i: (0, 0)),
    ],
    out_specs=[
        pl.BlockSpec((BLK, 128), lambda i: (i, 0)),
        pl.BlockSpec((BLK, 128), lambda i: (i, 0)),
        pl.BlockSpec((BLK, 128), lambda i: (i, 0)),
    ],
    out_shape=[
        jax.ShapeDtypeStruct((NP, 128), jnp.float32),
        jax.ShapeDtypeStruct((NP, 128), jnp.float32),
        jax.ShapeDtypeStruct((NP, 128), jnp.float32),
    ],
)


def _k4_body(dinv_ref, a1a_ref, a1b_ref, ha_ref, hb_ref, b1_ref, w2_ref,
             h2p_ref):
    dv = dinv_ref[...]
    h1a = jnp.maximum(dv * (a1a_ref[...] + ha_ref[...]) + b1_ref[0, :128], 0.0)
    h1b = jnp.maximum(dv * (a1b_ref[...] + hb_ref[...]) + b1_ref[0, 128:], 0.0)
    h2 = jnp.dot(h1a, w2_ref[:128], preferred_element_type=jnp.float32)
    h2 = h2 + jnp.dot(h1b, w2_ref[128:], preferred_element_type=jnp.float32)
    h2p_ref[...] = jnp.concatenate(
        [h2 * dv[:, :OUT], jnp.zeros((BLK, 128 - OUT), jnp.float32)], axis=1
    )


_k4 = pl.pallas_call(
    _k4_body,
    grid=(GRID,),
    in_specs=[
        pl.BlockSpec((BLK, 128), lambda i: (i, 0)),
        pl.BlockSpec((BLK, 128), lambda i: (i, 0)),
        pl.BlockSpec((BLK, 128), lambda i: (i, 0)),
        pl.BlockSpec((BLK, 128), lambda i: (i, 0)),
        pl.BlockSpec((BLK, 128), lambda i: (i, 0)),
        pl.BlockSpec((1, HID), lambda i: (0, 0)),
        pl.BlockSpec((HID, OUT), lambda i: (0, 0)),
    ],
    out_specs=pl.BlockSpec((BLK, 128), lambda i: (i, 0)),
    out_shape=jax.ShapeDtypeStruct((NP, 128), jnp.float32),
)


def _k6_body(dinv_ref, p0_ref, p1_ref, h2p_ref, b2_ref, out_ref):
    out_ref[...] = (
        dinv_ref[:, :OUT]
        * (p0_ref[:, :OUT] + p1_ref[:, :OUT] + h2p_ref[:, :OUT])
        + b2_ref[0, :]
    )


_k6 = pl.pallas_call(
    _k6_body,
    grid=(GRID,),
    in_specs=[
        pl.BlockSpec((BLK, 128), lambda i: (i, 0)),
        pl.BlockSpec((BLK, 128), lambda i: (i, 0)),
        pl.BlockSpec((BLK, 128), lambda i: (i, 0)),
        pl.BlockSpec((BLK, 128), lambda i: (i, 0)),
        pl.BlockSpec((1, OUT), lambda i: (0, 0)),
    ],
    out_specs=pl.BlockSpec((BLK, OUT), lambda i: (i, 0)),
    out_shape=jax.ShapeDtypeStruct((NP, OUT), jnp.float32),
)


@jax.jit
def kernel(x, edge_index, W1, b1, W2, b2):
    ei = edge_index.astype(jnp.int32)
    pad = jnp.full((EP - E,), N, dtype=jnp.int32)
    src = jnp.concatenate([ei[0], pad])
    dst = jnp.concatenate([ei[1], pad])
    xp = jnp.zeros((NP, IN_DIM), jnp.float32).at[:N].set(x)
    zeros128 = jnp.zeros((NP, 128), jnp.float32)
    ones128 = jnp.ones((CH, 128), jnp.float32)

    degp = _k1_deg(dst, zeros128, ones128)
    dinv, ha, hb = _k2(degp[0], degp[1], xp, W1)
    a1 = _k3_agg1(src, dst, ha, hb, zeros128)
    h2p = _k4(dinv, a1[0], a1[1], ha, hb, b1.reshape(1, HID), W2)
    a2 = _k5_agg2(src, dst, h2p, zeros128)
    out = _k6(dinv, a2[0], a2[1], h2p, b2.reshape(1, OUT))
    return out[:N]


# speedup vs baseline: 6.7659x; 1.1414x over previous
import functools

import jax
import jax.numpy as jnp
from jax import lax
from jax.experimental import pallas as pl
from jax.experimental.pallas import tpu as pltpu
from jax.experimental.pallas import tpu_sc as plsc

N = 10000
E = 160000
IN_DIM = 256
HID = 256
OUT = 64

NP = 10240
EP = 163840
CH = 128
NTILES = 16
RPT = NP // NTILES

_mesh = plsc.VectorSubcoreMesh(core_axis_name="c", subcore_axis_name="s")


@functools.partial(
    pl.kernel,
    out_type=jax.ShapeDtypeStruct((2, NP, 128), jnp.float32),
    mesh=_mesh,
    scratch_types=[
        pltpu.VMEM((EP // 32 // CH, CH), jnp.int32),
        pltpu.VMEM((CH, 128), jnp.float32),
        pltpu.VMEM_SHARED((NP, 128), jnp.float32),
        pltpu.SemaphoreType.DMA,
    ],
)
def _k1_deg(dst_hbm, zeros_hbm, ones_hbm, out_hbm, dst_all, ones_v, acc, sem):
    cid = lax.axis_index("c")
    sid = lax.axis_index("s")
    nch = EP // 32 // CH
    pltpu.sync_copy(
        zeros_hbm.at[pl.ds(sid * RPT, RPT)],
        acc.at[pl.ds(sid * RPT, RPT)],
    )
    wid = cid * NTILES + sid
    pltpu.sync_copy(dst_hbm.at[pl.ds(wid * nch, nch)], dst_all)
    pltpu.sync_copy(ones_hbm, ones_v)
    plsc.subcore_barrier()

    def fire(c, carry):
        pltpu.async_copy(ones_v, acc.at[dst_all.at[c]], sem, add=True)
        return carry

    lax.fori_loop(0, nch, fire, 0)

    def drain(c, carry):
        pltpu.make_async_copy(ones_hbm, ones_v, sem).wait()
        return carry

    lax.fori_loop(0, nch, drain, 0)
    plsc.subcore_barrier()
    pltpu.sync_copy(
        acc.at[pl.ds(sid * RPT, RPT)],
        out_hbm.at[cid, pl.ds(sid * RPT, RPT)],
    )


NBUF = 2
SEC = 40


def _agg_pipeline(tbl, acc, src_hbm, dst_hbm, chunk0, nch,
                  src_all, dst_all, rows, gsems, ssems):
    def wait_gather(b):
        pltpu.make_async_copy(tbl.at[src_all.at[0]], rows.at[b], gsems[b]).wait()

    def wait_scatter(b):
        pltpu.make_async_copy(tbl.at[src_all.at[0]], rows.at[b], ssems[b]).wait()

    for sec in range(nch // SEC):
        pltpu.sync_copy(src_hbm.at[pl.ds(chunk0 + sec * SEC, SEC)], src_all)
        pltpu.sync_copy(dst_hbm.at[pl.ds(chunk0 + sec * SEC, SEC)], dst_all)
        for b in range(NBUF):
            pltpu.async_copy(tbl.at[src_all.at[b]], rows.at[b], gsems[b])

        def body(g, carry):
            for b in range(NBUF):
                wait_gather(b)
                pltpu.async_copy(
                    rows.at[b], acc.at[dst_all.at[g * NBUF + b]], ssems[b],
                    add=True,
                )
            for b in range(NBUF):
                wait_scatter(b)
                nxt = g * NBUF + b + NBUF

                @pl.when(nxt < SEC)
                def _():
                    pltpu.async_copy(
                        tbl.at[src_all.at[nxt]], rows.at[b], gsems[b]
                    )
            return carry

        lax.fori_loop(0, SEC // NBUF, body, 0)


@functools.partial(
    pl.kernel,
    out_type=jax.ShapeDtypeStruct((2, NP, 128), jnp.float32),
    mesh=_mesh,
    scratch_types=[
        pltpu.VMEM((SEC, CH), jnp.int32),
        pltpu.VMEM((SEC, CH), jnp.int32),
        pltpu.VMEM((NBUF, CH, 128), jnp.float32),
        pltpu.VMEM_SHARED((NP, 128), jnp.float32),
    ]
    + [pltpu.SemaphoreType.DMA] * (2 * NBUF),
)
def _k3_agg1(src_hbm, dst_hbm, ha_hbm, hb_hbm, zeros_hbm, out_hbm,
             src_all, dst_all, rows, acc, *sems):
    gsems, ssems = sems[:NBUF], sems[NBUF:]
    cid = lax.axis_index("c")
    sid = lax.axis_index("s")
    nch = EP // NTILES // CH
    pltpu.sync_copy(
        zeros_hbm.at[pl.ds(sid * RPT, RPT)], acc.at[pl.ds(sid * RPT, RPT)]
    )
    plsc.subcore_barrier()

    @pl.when(cid == 0)
    def _():
        _agg_pipeline(ha_hbm, acc, src_hbm, dst_hbm, sid * nch, nch,
                      src_all, dst_all, rows, gsems, ssems)

    @pl.when(cid == 1)
    def _():
        _agg_pipeline(hb_hbm, acc, src_hbm, dst_hbm, sid * nch, nch,
                      src_all, dst_all, rows, gsems, ssems)

    plsc.subcore_barrier()
    pltpu.sync_copy(
        acc.at[pl.ds(sid * RPT, RPT)],
        out_hbm.at[cid, pl.ds(sid * RPT, RPT)],
    )


@functools.partial(
    pl.kernel,
    out_type=jax.ShapeDtypeStruct((2, NP, 128), jnp.float32),
    mesh=_mesh,
    scratch_types=[
        pltpu.VMEM((SEC, CH), jnp.int32),
        pltpu.VMEM((SEC, CH), jnp.int32),
        pltpu.VMEM((NBUF, CH, 128), jnp.float32),
        pltpu.VMEM_SHARED((NP, 128), jnp.float32),
    ]
    + [pltpu.SemaphoreType.DMA] * (2 * NBUF),
)
def _k5_agg2(src_hbm, dst_hbm, h2p_hbm, zeros_hbm, out_hbm,
             src_all, dst_all, rows, acc, *sems):
    gsems, ssems = sems[:NBUF], sems[NBUF:]
    cid = lax.axis_index("c")
    sid = lax.axis_index("s")
    nch = EP // 32 // CH
    pltpu.sync_copy(
        zeros_hbm.at[pl.ds(sid * RPT, RPT)],
        acc.at[pl.ds(sid * RPT, RPT)],
    )
    wid = cid * NTILES + sid
    plsc.subcore_barrier()
    _agg_pipeline(h2p_hbm, acc, src_hbm, dst_hbm, wid * nch, nch,
                  src_all, dst_all, rows, gsems, ssems)
    plsc.subcore_barrier()
    pltpu.sync_copy(
        acc.at[pl.ds(sid * RPT, RPT)],
        out_hbm.at[cid, pl.ds(sid * RPT, RPT)],
    )


BLK = 256
GRID = NP // BLK


def _k2_body(deg0_ref, deg1_ref, x_ref, w1_ref, dinv_ref, ha_ref, hb_ref):
    deg = deg0_ref[:, 0] + deg1_ref[:, 0] + 1.0
    dinv = lax.rsqrt(deg)[:, None]
    h = jnp.dot(x_ref[...], w1_ref[...], preferred_element_type=jnp.float32)
    dinv_ref[...] = jnp.broadcast_to(dinv, (BLK, 128))
    ha_ref[...] = h[:, :128] * dinv
    hb_ref[...] = h[:, 128:] * dinv


_k2 = pl.pallas_call(
    _k2_body,
    grid=(GRID,),
    in_specs=[
        pl.BlockSpec((BLK, 128), lambda i: (i, 0)),
        pl.BlockSpec((BLK, 128), lambda i: (i, 0)),
        pl.BlockSpec((BLK, IN_DIM), lambda i: (i, 0)),
        pl.BlockSpec((IN_DIM, HID), lambda i: (0, 0)),
    ],
    out_specs=[
        pl.BlockSpec((BLK, 128), lambda i: (i, 0)),
        pl.BlockSpec((BLK, 128), lambda i: (i, 0)),
        pl.BlockSpec((BLK, 128), lambda i: (i, 0)),
    ],
    out_shape=[
        jax.ShapeDtypeStruct((NP, 128), jnp.float32),
        jax.ShapeDtypeStruct((NP, 128), jnp.float32),
        jax.ShapeDtypeStruct((NP, 128), jnp.float32),
    ],
)


def _k4_body(dinv_ref, a1a_ref, a1b_ref, ha_ref, hb_ref, b1_ref, w2_ref,
             h2p_ref):
    dv = dinv_ref[...]
    h1a = jnp.maximum(dv * (a1a_ref[...] + ha_ref[...]) + b1_ref[0, :128], 0.0)
    h1b = jnp.maximum(dv * (a1b_ref[...] + hb_ref[...]) + b1_ref[0, 128:], 0.0)
    h2 = jnp.dot(h1a, w2_ref[:128], preferred_element_type=jnp.float32)
    h2 = h2 + jnp.dot(h1b, w2_ref[128:], preferred_element_type=jnp.float32)
    h2p_ref[...] = jnp.concatenate(
        [h2 * dv[:, :OUT], jnp.zeros((BLK, 128 - OUT), jnp.float32)], axis=1
    )


_k4 = pl.pallas_call(
    _k4_body,
    grid=(GRID,),
    in_specs=[
        pl.BlockSpec((BLK, 128), lambda i: (i, 0)),
        pl.BlockSpec((BLK, 128), lambda i: (i, 0)),
        pl.BlockSpec((BLK, 128), lambda i: (i, 0)),
        pl.BlockSpec((BLK, 128), lambda i: (i, 0)),
        pl.BlockSpec((BLK, 128), lambda i: (i, 0)),
        pl.BlockSpec((1, HID), lambda i: (0, 0)),
        pl.BlockSpec((HID, OUT), lambda i: (0, 0)),
    ],
    out_specs=pl.BlockSpec((BLK, 128), lambda i: (i, 0)),
    out_shape=jax.ShapeDtypeStruct((NP, 128), jnp.float32),
)


def _k6_body(dinv_ref, p0_ref, p1_ref, h2p_ref, b2_ref, out_ref):
    out_ref[...] = (
        dinv_ref[:, :OUT]
        * (p0_ref[:, :OUT] + p1_ref[:, :OUT] + h2p_ref[:, :OUT])
        + b2_ref[0, :]
    )


_k6 = pl.pallas_call(
    _k6_body,
    grid=(GRID,),
    in_specs=[
        pl.BlockSpec((BLK, 128), lambda i: (i, 0)),
        pl.BlockSpec((BLK, 128), lambda i: (i, 0)),
        pl.BlockSpec((BLK, 128), lambda i: (i, 0)),
        pl.BlockSpec((BLK, 128), lambda i: (i, 0)),
        pl.BlockSpec((1, OUT), lambda i: (0, 0)),
    ],
    out_specs=pl.BlockSpec((BLK, OUT), lambda i: (i, 0)),
    out_shape=jax.ShapeDtypeStruct((NP, OUT), jnp.float32),
)


@jax.jit
def kernel(x, edge_index, W1, b1, W2, b2):
    ei = edge_index.astype(jnp.int32)
    pad = jnp.full((EP - E,), N, dtype=jnp.int32)
    src = jnp.concatenate([ei[0], pad]).reshape(EP // CH, CH)
    dst = jnp.concatenate([ei[1], pad]).reshape(EP // CH, CH)
    xp = jnp.zeros((NP, IN_DIM), jnp.float32).at[:N].set(x)
    zeros128 = jnp.zeros((NP, 128), jnp.float32)
    ones128 = jnp.ones((CH, 128), jnp.float32)

    degp = _k1_deg(dst, zeros128, ones128)
    dinv, ha, hb = _k2(degp[0], degp[1], xp, W1)
    a1 = _k3_agg1(src, dst, ha, hb, zeros128)
    h2p = _k4(dinv, a1[0], a1[1], ha, hb, b1.reshape(1, HID), W2)
    a2 = _k5_agg2(src, dst, h2p, zeros128)
    out = _k6(dinv, a2[0], a2[1], h2p, b2.reshape(1, OUT))
    return out[:N]


# speedup vs baseline: 7.1152x; 1.0516x over previous
import functools

import jax
import jax.numpy as jnp
from jax import lax
from jax.experimental import pallas as pl
from jax.experimental.pallas import tpu as pltpu
from jax.experimental.pallas import tpu_sc as plsc

N = 10000
E = 160000
IN_DIM = 256
HID = 256
OUT = 64

NP = 10240
EP = 163840
CH = 64
NTILES = 16
RPT = NP // NTILES

_mesh = plsc.VectorSubcoreMesh(core_axis_name="c", subcore_axis_name="s")


@functools.partial(
    pl.kernel,
    out_type=jax.ShapeDtypeStruct((2, NP, 128), jnp.float32),
    mesh=_mesh,
    scratch_types=[
        pltpu.VMEM((EP // 32 // CH, CH), jnp.int32),
        pltpu.VMEM((CH, 128), jnp.float32),
        pltpu.VMEM_SHARED((NP, 128), jnp.float32),
        pltpu.SemaphoreType.DMA,
    ],
)
def _k1_deg(dst_hbm, zeros_hbm, ones_hbm, out_hbm, dst_all, ones_v, acc, sem):
    cid = lax.axis_index("c")
    sid = lax.axis_index("s")
    nch = EP // 32 // CH
    pltpu.sync_copy(
        zeros_hbm.at[pl.ds(sid * RPT, RPT)],
        acc.at[pl.ds(sid * RPT, RPT)],
    )
    wid = cid * NTILES + sid
    pltpu.sync_copy(dst_hbm.at[pl.ds(wid * nch, nch)], dst_all)
    pltpu.sync_copy(ones_hbm, ones_v)
    plsc.subcore_barrier()

    def fire(c, carry):
        pltpu.async_copy(ones_v, acc.at[dst_all.at[c]], sem, add=True)
        return carry

    lax.fori_loop(0, nch, fire, 0)

    def drain(c, carry):
        pltpu.make_async_copy(ones_hbm, ones_v, sem).wait()
        return carry

    lax.fori_loop(0, nch, drain, 0)
    plsc.subcore_barrier()
    pltpu.sync_copy(
        acc.at[pl.ds(sid * RPT, RPT)],
        out_hbm.at[cid, pl.ds(sid * RPT, RPT)],
    )


NBUF = 4
SEC = 40


def _agg_pipeline(tbl, acc, src_hbm, dst_hbm, chunk0, nch,
                  src_all, dst_all, rows, gsems, ssems):
    def wait_gather(b):
        pltpu.make_async_copy(tbl.at[src_all.at[0]], rows.at[b], gsems[b]).wait()

    def wait_scatter(b):
        pltpu.make_async_copy(tbl.at[src_all.at[0]], rows.at[b], ssems[b]).wait()

    for sec in range(nch // SEC):
        pltpu.sync_copy(src_hbm.at[pl.ds(chunk0 + sec * SEC, SEC)], src_all)
        pltpu.sync_copy(dst_hbm.at[pl.ds(chunk0 + sec * SEC, SEC)], dst_all)
        for b in range(NBUF):
            pltpu.async_copy(tbl.at[src_all.at[b]], rows.at[b], gsems[b])

        def body(g, carry):
            for b in range(NBUF):
                wait_gather(b)
                pltpu.async_copy(
                    rows.at[b], acc.at[dst_all.at[g * NBUF + b]], ssems[b],
                    add=True,
                )
            for b in range(NBUF):
                wait_scatter(b)
                nxt = g * NBUF + b + NBUF

                @pl.when(nxt < SEC)
                def _():
                    pltpu.async_copy(
                        tbl.at[src_all.at[nxt]], rows.at[b], gsems[b]
                    )
            return carry

        lax.fori_loop(0, SEC // NBUF, body, 0)


@functools.partial(
    pl.kernel,
    out_type=jax.ShapeDtypeStruct((2, NP, 128), jnp.float32),
    mesh=_mesh,
    scratch_types=[
        pltpu.VMEM((SEC, CH), jnp.int32),
        pltpu.VMEM((SEC, CH), jnp.int32),
        pltpu.VMEM((NBUF, CH, 128), jnp.float32),
        pltpu.VMEM_SHARED((NP, 128), jnp.float32),
    ]
    + [pltpu.SemaphoreType.DMA] * (2 * NBUF),
)
def _k3_agg1(src_hbm, dst_hbm, ha_hbm, hb_hbm, zeros_hbm, out_hbm,
             src_all, dst_all, rows, acc, *sems):
    gsems, ssems = sems[:NBUF], sems[NBUF:]
    cid = lax.axis_index("c")
    sid = lax.axis_index("s")
    nch = EP // NTILES // CH
    pltpu.sync_copy(
        zeros_hbm.at[pl.ds(sid * RPT, RPT)], acc.at[pl.ds(sid * RPT, RPT)]
    )
    plsc.subcore_barrier()

    @pl.when(cid == 0)
    def _():
        _agg_pipeline(ha_hbm, acc, src_hbm, dst_hbm, sid * nch, nch,
                      src_all, dst_all, rows, gsems, ssems)

    @pl.when(cid == 1)
    def _():
        _agg_pipeline(hb_hbm, acc, src_hbm, dst_hbm, sid * nch, nch,
                      src_all, dst_all, rows, gsems, ssems)

    plsc.subcore_barrier()
    pltpu.sync_copy(
        acc.at[pl.ds(sid * RPT, RPT)],
        out_hbm.at[cid, pl.ds(sid * RPT, RPT)],
    )


@functools.partial(
    pl.kernel,
    out_type=jax.ShapeDtypeStruct((2, NP, 128), jnp.float32),
    mesh=_mesh,
    scratch_types=[
        pltpu.VMEM((SEC, CH), jnp.int32),
        pltpu.VMEM((SEC, CH), jnp.int32),
        pltpu.VMEM((NBUF, CH, 128), jnp.float32),
        pltpu.VMEM_SHARED((NP, 128), jnp.float32),
    ]
    + [pltpu.SemaphoreType.DMA] * (2 * NBUF),
)
def _k5_agg2(src_hbm, dst_hbm, h2p_hbm, zeros_hbm, out_hbm,
             src_all, dst_all, rows, acc, *sems):
    gsems, ssems = sems[:NBUF], sems[NBUF:]
    cid = lax.axis_index("c")
    sid = lax.axis_index("s")
    nch = EP // 32 // CH
    pltpu.sync_copy(
        zeros_hbm.at[pl.ds(sid * RPT, RPT)],
        acc.at[pl.ds(sid * RPT, RPT)],
    )
    wid = cid * NTILES + sid
    plsc.subcore_barrier()
    _agg_pipeline(h2p_hbm, acc, src_hbm, dst_hbm, wid * nch, nch,
                  src_all, dst_all, rows, gsems, ssems)
    plsc.subcore_barrier()
    pltpu.sync_copy(
        acc.at[pl.ds(sid * RPT, RPT)],
        out_hbm.at[cid, pl.ds(sid * RPT, RPT)],
    )


BLK = 256
GRID = NP // BLK


def _k2_body(deg0_ref, deg1_ref, x_ref, w1_ref, dinv_ref, ha_ref, hb_ref):
    deg = deg0_ref[:, 0] + deg1_ref[:, 0] + 1.0
    dinv = lax.rsqrt(deg)[:, None]
    h = jnp.dot(x_ref[...], w1_ref[...], preferred_element_type=jnp.float32)
    dinv_ref[...] = jnp.broadcast_to(dinv, (BLK, 128))
    ha_ref[...] = h[:, :128] * dinv
    hb_ref[...] = h[:, 128:] * dinv


_k2 = pl.pallas_call(
    _k2_body,
    grid=(GRID,),
    in_specs=[
        pl.BlockSpec((BLK, 128), lambda i: (i, 0)),
        pl.BlockSpec((BLK, 128), lambda i: (i, 0)),
        pl.BlockSpec((BLK, IN_DIM), lambda i: (i, 0)),
        pl.BlockSpec((IN_DIM, HID), lambda i: (0, 0)),
    ],
    out_specs=[
        pl.BlockSpec((BLK, 128), lambda i: (i, 0)),
        pl.BlockSpec((BLK, 128), lambda i: (i, 0)),
        pl.BlockSpec((BLK, 128), lambda i: (i, 0)),
    ],
    out_shape=[
        jax.ShapeDtypeStruct((NP, 128), jnp.float32),
        jax.ShapeDtypeStruct((NP, 128), jnp.float32),
        jax.ShapeDtypeStruct((NP, 128), jnp.float32),
    ],
)


def _k4_body(dinv_ref, a1a_ref, a1b_ref, ha_ref, hb_ref, b1_ref, w2_ref,
             h2p_ref):
    dv = dinv_ref[...]
    h1a = jnp.maximum(dv * (a1a_ref[...] + ha_ref[...]) + b1_ref[0, :128], 0.0)
    h1b = jnp.maximum(dv * (a1b_ref[...] + hb_ref[...]) + b1_ref[0, 128:], 0.0)
    h2 = jnp.dot(h1a, w2_ref[:128], preferred_element_type=jnp.float32)
    h2 = h2 + jnp.dot(h1b, w2_ref[128:], preferred_element_type=jnp.float32)
    h2p_ref[...] = jnp.concatenate(
        [h2 * dv[:, :OUT], jnp.zeros((BLK, 128 - OUT), jnp.float32)], axis=1
    )


_k4 = pl.pallas_call(
    _k4_body,
    grid=(GRID,),
    in_specs=[
        pl.BlockSpec((BLK, 128), lambda i: (i, 0)),
        pl.BlockSpec((BLK, 128), lambda i: (i, 0)),
        pl.BlockSpec((BLK, 128), lambda i: (i, 0)),
        pl.BlockSpec((BLK, 128), lambda i: (i, 0)),
        pl.BlockSpec((BLK, 128), lambda i: (i, 0)),
        pl.BlockSpec((1, HID), lambda i: (0, 0)),
        pl.BlockSpec((HID, OUT), lambda i: (0, 0)),
    ],
    out_specs=pl.BlockSpec((BLK, 128), lambda i: (i, 0)),
    out_shape=jax.ShapeDtypeStruct((NP, 128), jnp.float32),
)


def _k6_body(dinv_ref, p0_ref, p1_ref, h2p_ref, b2_ref, out_ref):
    out_ref[...] = (
        dinv_ref[:, :OUT]
        * (p0_ref[:, :OUT] + p1_ref[:, :OUT] + h2p_ref[:, :OUT])
        + b2_ref[0, :]
    )


_k6 = pl.pallas_call(
    _k6_body,
    grid=(GRID,),
    in_specs=[
        pl.BlockSpec((BLK, 128), lambda i: (i, 0)),
        pl.BlockSpec((BLK, 128), lambda i: (i, 0)),
        pl.BlockSpec((BLK, 128), lambda i: (i, 0)),
        pl.BlockSpec((BLK, 128), lambda i: (i, 0)),
        pl.BlockSpec((1, OUT), lambda i: (0, 0)),
    ],
    out_specs=pl.BlockSpec((BLK, OUT), lambda i: (i, 0)),
    out_shape=jax.ShapeDtypeStruct((NP, OUT), jnp.float32),
)


@jax.jit
def kernel(x, edge_index, W1, b1, W2, b2):
    ei = edge_index.astype(jnp.int32)
    pad = jnp.full((EP - E,), N, dtype=jnp.int32)
    src = jnp.concatenate([ei[0], pad]).reshape(EP // CH, CH)
    dst = jnp.concatenate([ei[1], pad]).reshape(EP // CH, CH)
    xp = jnp.zeros((NP, IN_DIM), jnp.float32).at[:N].set(x)
    zeros128 = jnp.zeros((NP, 128), jnp.float32)
    ones128 = jnp.ones((CH, 128), jnp.float32)

    degp = _k1_deg(dst, zeros128, ones128)
    dinv, ha, hb = _k2(degp[0], degp[1], xp, W1)
    a1 = _k3_agg1(src, dst, ha, hb, zeros128)
    h2p = _k4(dinv, a1[0], a1[1], ha, hb, b1.reshape(1, HID), W2)
    a2 = _k5_agg2(src, dst, h2p, zeros128)
    out = _k6(dinv, a2[0], a2[1], h2p, b2.reshape(1, OUT))
    return out[:N]


# speedup vs baseline: 7.8314x; 1.1007x over previous
import functools

import jax
import jax.numpy as jnp
from jax import lax
from jax.experimental import pallas as pl
from jax.experimental.pallas import tpu as pltpu
from jax.experimental.pallas import tpu_sc as plsc

N = 10000
E = 160000
IN_DIM = 256
HID = 256
OUT = 64

NP = 10240
EP = 163840
CH = 64
NTILES = 16
RPT = NP // NTILES

_mesh = plsc.VectorSubcoreMesh(core_axis_name="c", subcore_axis_name="s")


@functools.partial(
    pl.kernel,
    out_type=jax.ShapeDtypeStruct((2, NP, 128), jnp.float32),
    mesh=_mesh,
    scratch_types=[
        pltpu.VMEM((EP // 32 // CH, CH), jnp.int32),
        pltpu.VMEM((CH, 128), jnp.float32),
        pltpu.VMEM_SHARED((NP, 128), jnp.float32),
        pltpu.SemaphoreType.DMA,
    ],
)
def _k1_deg(dst_hbm, zeros_hbm, ones_hbm, out_hbm, dst_all, ones_v, acc, sem):
    cid = lax.axis_index("c")
    sid = lax.axis_index("s")
    nch = EP // 32 // CH
    pltpu.sync_copy(
        zeros_hbm.at[pl.ds(sid * RPT, RPT)],
        acc.at[pl.ds(sid * RPT, RPT)],
    )
    wid = cid * NTILES + sid
    pltpu.sync_copy(dst_hbm.at[pl.ds(wid * nch, nch)], dst_all)
    pltpu.sync_copy(ones_hbm, ones_v)
    plsc.subcore_barrier()

    def fire(c, carry):
        pltpu.async_copy(ones_v, acc.at[dst_all.at[c]], sem, add=True)
        return carry

    lax.fori_loop(0, nch, fire, 0)

    def drain(c, carry):
        pltpu.make_async_copy(ones_hbm, ones_v, sem).wait()
        return carry

    lax.fori_loop(0, nch, drain, 0)
    plsc.subcore_barrier()
    pltpu.sync_copy(
        acc.at[pl.ds(sid * RPT, RPT)],
        out_hbm.at[cid, pl.ds(sid * RPT, RPT)],
    )


NBUF = 4
SEC = 40


def _agg_pipeline(tbl, acc, src_hbm, dst_hbm, chunk0, nch,
                  src_all, dst_all, rows, gsems, ssems):
    def wait_gather(b):
        pltpu.make_async_copy(tbl.at[src_all.at[0]], rows.at[b], gsems[b]).wait()

    def wait_scatter(b):
        pltpu.make_async_copy(tbl.at[src_all.at[0]], rows.at[b], ssems[b]).wait()

    for sec in range(nch // SEC):
        pltpu.sync_copy(src_hbm.at[pl.ds(chunk0 + sec * SEC, SEC)], src_all)
        pltpu.sync_copy(dst_hbm.at[pl.ds(chunk0 + sec * SEC, SEC)], dst_all)
        for b in range(NBUF):
            pltpu.async_copy(tbl.at[src_all.at[b]], rows.at[b], gsems[b])

        def body(g, carry):
            for b in range(NBUF):
                wait_gather(b)
                pltpu.async_copy(
                    rows.at[b], acc.at[dst_all.at[g * NBUF + b]], ssems[b],
                    add=True,
                )
            for b in range(NBUF):
                wait_scatter(b)
                nxt = g * NBUF + b + NBUF

                @pl.when(nxt < SEC)
                def _():
                    pltpu.async_copy(
                        tbl.at[src_all.at[nxt]], rows.at[b], gsems[b]
                    )
            return carry

        lax.fori_loop(0, SEC // NBUF, body, 0)


@functools.partial(
    pl.kernel,
    out_type=jax.ShapeDtypeStruct((2, NP, 128), jnp.float32),
    mesh=_mesh,
    scratch_types=[
        pltpu.VMEM((SEC, CH), jnp.int32),
        pltpu.VMEM((SEC, CH), jnp.int32),
        pltpu.VMEM((NBUF, CH, 128), jnp.float32),
        pltpu.VMEM_SHARED((NP, 128), jnp.float32),
    ]
    + [pltpu.SemaphoreType.DMA] * (2 * NBUF),
)
def _k3_agg1(src_hbm, dst_hbm, ha_hbm, hb_hbm, zeros_hbm, out_hbm,
             src_all, dst_all, rows, acc, *sems):
    gsems, ssems = sems[:NBUF], sems[NBUF:]
    cid = lax.axis_index("c")
    sid = lax.axis_index("s")
    nch = EP // NTILES // CH
    pltpu.sync_copy(
        zeros_hbm.at[pl.ds(sid * RPT, RPT)], acc.at[pl.ds(sid * RPT, RPT)]
    )
    plsc.subcore_barrier()

    @pl.when(cid == 0)
    def _():
        _agg_pipeline(ha_hbm, acc, src_hbm, dst_hbm, sid * nch, nch,
                      src_all, dst_all, rows, gsems, ssems)

    @pl.when(cid == 1)
    def _():
        _agg_pipeline(hb_hbm, acc, src_hbm, dst_hbm, sid * nch, nch,
                      src_all, dst_all, rows, gsems, ssems)

    plsc.subcore_barrier()
    pltpu.sync_copy(
        acc.at[pl.ds(sid * RPT, RPT)],
        out_hbm.at[cid, pl.ds(sid * RPT, RPT)],
    )


@functools.partial(
    pl.kernel,
    out_type=jax.ShapeDtypeStruct((2, NP, 128), jnp.float32),
    mesh=_mesh,
    scratch_types=[
        pltpu.VMEM((SEC, CH), jnp.int32),
        pltpu.VMEM((SEC, CH), jnp.int32),
        pltpu.VMEM((NBUF, CH, 128), jnp.float32),
        pltpu.VMEM_SHARED((NP, 128), jnp.float32),
    ]
    + [pltpu.SemaphoreType.DMA] * (2 * NBUF),
)
def _k5_agg2(src_hbm, dst_hbm, h2p_hbm, zeros_hbm, out_hbm,
             src_all, dst_all, rows, acc, *sems):
    gsems, ssems = sems[:NBUF], sems[NBUF:]
    cid = lax.axis_index("c")
    sid = lax.axis_index("s")
    nch = EP // 32 // CH
    pltpu.sync_copy(
        zeros_hbm.at[pl.ds(sid * RPT, RPT)],
        acc.at[pl.ds(sid * RPT, RPT)],
    )
    wid = cid * NTILES + sid
    plsc.subcore_barrier()
    _agg_pipeline(h2p_hbm, acc, src_hbm, dst_hbm, wid * nch, nch,
                  src_all, dst_all, rows, gsems, ssems)
    plsc.subcore_barrier()
    pltpu.sync_copy(
        acc.at[pl.ds(sid * RPT, RPT)],
        out_hbm.at[cid, pl.ds(sid * RPT, RPT)],
    )


BLK = 256
GRID = NP // BLK


def _k2a_body(x_ref, w1_ref, ha_ref, hb_ref):
    h = jnp.dot(x_ref[...], w1_ref[...], preferred_element_type=jnp.float32)
    ha_ref[...] = h[:, :128]
    hb_ref[...] = h[:, 128:]


_k2a = pl.pallas_call(
    _k2a_body,
    grid=(GRID,),
    in_specs=[
        pl.BlockSpec((BLK, IN_DIM), lambda i: (i, 0)),
        pl.BlockSpec((IN_DIM, HID), lambda i: (0, 0)),
    ],
    out_specs=[
        pl.BlockSpec((BLK, 128), lambda i: (i, 0)),
        pl.BlockSpec((BLK, 128), lambda i: (i, 0)),
    ],
    out_shape=[
        jax.ShapeDtypeStruct((NP, 128), jnp.float32),
        jax.ShapeDtypeStruct((NP, 128), jnp.float32),
    ],
)


def _k2b_body(deg0_ref, deg1_ref, hra_ref, hrb_ref, dinv_ref, ha_ref, hb_ref):
    deg = deg0_ref[:, 0] + deg1_ref[:, 0] + 1.0
    dinv = lax.rsqrt(deg)[:, None]
    dinv_ref[...] = jnp.broadcast_to(dinv, (BLK, 128))
    ha_ref[...] = hra_ref[...] * dinv
    hb_ref[...] = hrb_ref[...] * dinv


_k2b = pl.pallas_call(
    _k2b_body,
    grid=(GRID,),
    in_specs=[
        pl.BlockSpec((BLK, 128), lambda i: (i, 0)),
        pl.BlockSpec((BLK, 128), lambda i: (i, 0)),
        pl.BlockSpec((BLK, 128), lambda i: (i, 0)),
        pl.BlockSpec((BLK, 128), lambda i: (i, 0)),
    ],
    out_specs=[
        pl.BlockSpec((BLK, 128), lambda i: (i, 0)),
        pl.BlockSpec((BLK, 128), lambda i: (i, 0)),
        pl.BlockSpec((BLK, 128), lambda i: (i, 0)),
    ],
    out_shape=[
        jax.ShapeDtypeStruct((NP, 128), jnp.float32),
        jax.ShapeDtypeStruct((NP, 128), jnp.float32),
        jax.ShapeDtypeStruct((NP, 128), jnp.float32),
    ],
)


def _k4_body(dinv_ref, a1a_ref, a1b_ref, ha_ref, hb_ref, b1_ref, w2_ref,
             h2p_ref):
    dv = dinv_ref[...]
    h1a = jnp.maximum(dv * (a1a_ref[...] + ha_ref[...]) + b1_ref[0, :128], 0.0)
    h1b = jnp.maximum(dv * (a1b_ref[...] + hb_ref[...]) + b1_ref[0, 128:], 0.0)
    h2 = jnp.dot(h1a, w2_ref[:128], preferred_element_type=jnp.float32)
    h2 = h2 + jnp.dot(h1b, w2_ref[128:], preferred_element_type=jnp.float32)
    h2p_ref[...] = jnp.concatenate(
        [h2 * dv[:, :OUT], jnp.zeros((BLK, 128 - OUT), jnp.float32)], axis=1
    )


_k4 = pl.pallas_call(
    _k4_body,
    grid=(GRID,),
    in_specs=[
        pl.BlockSpec((BLK, 128), lambda i: (i, 0)),
        pl.BlockSpec((BLK, 128), lambda i: (i, 0)),
        pl.BlockSpec((BLK, 128), lambda i: (i, 0)),
        pl.BlockSpec((BLK, 128), lambda i: (i, 0)),
        pl.BlockSpec((BLK, 128), lambda i: (i, 0)),
        pl.BlockSpec((1, HID), lambda i: (0, 0)),
        pl.BlockSpec((HID, OUT), lambda i: (0, 0)),
    ],
    out_specs=pl.BlockSpec((BLK, 128), lambda i: (i, 0)),
    out_shape=jax.ShapeDtypeStruct((NP, 128), jnp.float32),
)


def _k6_body(dinv_ref, p0_ref, p1_ref, h2p_ref, b2_ref, out_ref):
    out_ref[...] = (
        dinv_ref[:, :OUT]
        * (p0_ref[:, :OUT] + p1_ref[:, :OUT] + h2p_ref[:, :OUT])
        + b2_ref[0, :]
    )


_k6 = pl.pallas_call(
    _k6_body,
    grid=(GRID,),
    in_specs=[
        pl.BlockSpec((BLK, 128), lambda i: (i, 0)),
        pl.BlockSpec((BLK, 128), lambda i: (i, 0)),
        pl.BlockSpec((BLK, 128), lambda i: (i, 0)),
        pl.BlockSpec((BLK, 128), lambda i: (i, 0)),
        pl.BlockSpec((1, OUT), lambda i: (0, 0)),
    ],
    out_specs=pl.BlockSpec((BLK, OUT), lambda i: (i, 0)),
    out_shape=jax.ShapeDtypeStruct((NP, OUT), jnp.float32),
)


@jax.jit
def kernel(x, edge_index, W1, b1, W2, b2):
    ei = edge_index.astype(jnp.int32)
    pad = jnp.full((EP - E,), N, dtype=jnp.int32)
    src = jnp.concatenate([ei[0], pad]).reshape(EP // CH, CH)
    dst = jnp.concatenate([ei[1], pad]).reshape(EP // CH, CH)
    xp = jnp.zeros((NP, IN_DIM), jnp.float32).at[:N].set(x)
    zeros128 = jnp.zeros((NP, 128), jnp.float32)
    ones128 = jnp.ones((CH, 128), jnp.float32)

    hra, hrb = _k2a(xp, W1)
    degp = _k1_deg(dst, zeros128, ones128)
    dinv, ha, hb = _k2b(degp[0], degp[1], hra, hrb)
    a1 = _k3_agg1(src, dst, ha, hb, zeros128)
    h2p = _k4(dinv, a1[0], a1[1], ha, hb, b1.reshape(1, HID), W2)
    a2 = _k5_agg2(src, dst, h2p, zeros128)
    out = _k6(dinv, a2[0], a2[1], h2p, b2.reshape(1, OUT))
    return out[:N]


# speedup vs baseline: 14.1115x; 1.8019x over previous
import functools

import jax
import jax.numpy as jnp
from jax import lax
from jax.experimental import pallas as pl
from jax.experimental.pallas import tpu as pltpu
from jax.experimental.pallas import tpu_sc as plsc

N = 10000
E = 160000
IN_DIM = 256
HID = 256
OUT = 64

NP = 10240
EP = 163840
CH = 64
NTILES = 16
RPT = NP // NTILES

_mesh = plsc.VectorSubcoreMesh(core_axis_name="c", subcore_axis_name="s")


@functools.partial(
    pl.kernel,
    out_type=jax.ShapeDtypeStruct((2, NP, 128), jnp.float32),
    mesh=_mesh,
    scratch_types=[
        pltpu.VMEM((EP // 32 // CH, CH), jnp.int32),
        pltpu.VMEM((CH, 128), jnp.float32),
        pltpu.VMEM_SHARED((NP, 128), jnp.float32),
        pltpu.SemaphoreType.DMA,
    ],
)
def _k1_deg(dst_hbm, zeros_hbm, ones_hbm, out_hbm, dst_all, ones_v, acc, sem):
    cid = lax.axis_index("c")
    sid = lax.axis_index("s")
    nch = EP // 32 // CH
    pltpu.sync_copy(
        zeros_hbm.at[pl.ds(sid * RPT, RPT)],
        acc.at[pl.ds(sid * RPT, RPT)],
    )
    wid = cid * NTILES + sid
    pltpu.sync_copy(dst_hbm.at[pl.ds(wid * nch, nch)], dst_all)
    pltpu.sync_copy(ones_hbm, ones_v)
    plsc.subcore_barrier()

    def fire(c, carry):
        pltpu.async_copy(ones_v, acc.at[dst_all.at[c]], sem, add=True)
        return carry

    lax.fori_loop(0, nch, fire, 0)

    def drain(c, carry):
        pltpu.make_async_copy(ones_hbm, ones_v, sem).wait()
        return carry

    lax.fori_loop(0, nch, drain, 0)
    plsc.subcore_barrier()
    pltpu.sync_copy(
        acc.at[pl.ds(sid * RPT, RPT)],
        out_hbm.at[cid, pl.ds(sid * RPT, RPT)],
    )


NBUF = 4
SEC = 40


def _agg_pipeline(tbl, acc, src_hbm, dst_hbm, chunk0, nch,
                  src_all, dst_all, rows, gsems, ssems):
    def wait_gather(b):
        pltpu.make_async_copy(tbl.at[src_all.at[0]], rows.at[b], gsems[b]).wait()

    def wait_scatter(b):
        pltpu.make_async_copy(tbl.at[src_all.at[0]], rows.at[b], ssems[b]).wait()

    for sec in range(nch // SEC):
        pltpu.sync_copy(src_hbm.at[pl.ds(chunk0 + sec * SEC, SEC)], src_all)
        pltpu.sync_copy(dst_hbm.at[pl.ds(chunk0 + sec * SEC, SEC)], dst_all)
        for b in range(NBUF):
            pltpu.async_copy(tbl.at[src_all.at[b]], rows.at[b], gsems[b])

        def body(g, carry):
            for b in range(NBUF):
                wait_gather(b)
                pltpu.async_copy(
                    rows.at[b], acc.at[dst_all.at[g * NBUF + b]], ssems[b],
                    add=True,
                )
            for b in range(NBUF):
                wait_scatter(b)
                nxt = g * NBUF + b + NBUF

                @pl.when(nxt < SEC)
                def _():
                    pltpu.async_copy(
                        tbl.at[src_all.at[nxt]], rows.at[b], gsems[b]
                    )
            return carry

        lax.fori_loop(0, SEC // NBUF, body, 0)


@functools.partial(
    pl.kernel,
    out_type=jax.ShapeDtypeStruct((2, NP, 128), jnp.float32),
    mesh=_mesh,
    scratch_types=[
        pltpu.VMEM((SEC, CH), jnp.int32),
        pltpu.VMEM((SEC, CH), jnp.int32),
        pltpu.VMEM((NBUF, CH, 128), jnp.float32),
        pltpu.VMEM_SHARED((NP, 128), jnp.float32),
    ]
    + [pltpu.SemaphoreType.DMA] * (2 * NBUF),
)
def _k3_agg1(src_hbm, dst_hbm, ha_hbm, hb_hbm, zeros_hbm, out_hbm,
             src_all, dst_all, rows, acc, *sems):
    gsems, ssems = sems[:NBUF], sems[NBUF:]
    cid = lax.axis_index("c")
    sid = lax.axis_index("s")
    nch = EP // NTILES // CH
    pltpu.sync_copy(
        zeros_hbm.at[pl.ds(sid * RPT, RPT)], acc.at[pl.ds(sid * RPT, RPT)]
    )
    plsc.subcore_barrier()

    @pl.when(cid == 0)
    def _():
        _agg_pipeline(ha_hbm, acc, src_hbm, dst_hbm, sid * nch, nch,
                      src_all, dst_all, rows, gsems, ssems)

    @pl.when(cid == 1)
    def _():
        _agg_pipeline(hb_hbm, acc, src_hbm, dst_hbm, sid * nch, nch,
                      src_all, dst_all, rows, gsems, ssems)

    plsc.subcore_barrier()
    pltpu.sync_copy(
        acc.at[pl.ds(sid * RPT, RPT)],
        out_hbm.at[cid, pl.ds(sid * RPT, RPT)],
    )


@functools.partial(
    pl.kernel,
    out_type=jax.ShapeDtypeStruct((2, NP, 128), jnp.float32),
    mesh=_mesh,
    scratch_types=[
        pltpu.VMEM((SEC, CH), jnp.int32),
        pltpu.VMEM((SEC, CH), jnp.int32),
        pltpu.VMEM((NBUF, CH, 128), jnp.float32),
        pltpu.VMEM_SHARED((NP, 128), jnp.float32),
    ]
    + [pltpu.SemaphoreType.DMA] * (2 * NBUF),
)
def _k5_agg2(src_hbm, dst_hbm, h2p_hbm, zeros_hbm, out_hbm,
             src_all, dst_all, rows, acc, *sems):
    gsems, ssems = sems[:NBUF], sems[NBUF:]
    cid = lax.axis_index("c")
    sid = lax.axis_index("s")
    nch = EP // 32 // CH
    pltpu.sync_copy(
        zeros_hbm.at[pl.ds(sid * RPT, RPT)],
        acc.at[pl.ds(sid * RPT, RPT)],
    )
    wid = cid * NTILES + sid
    plsc.subcore_barrier()
    _agg_pipeline(h2p_hbm, acc, src_hbm, dst_hbm, wid * nch, nch,
                  src_all, dst_all, rows, gsems, ssems)
    plsc.subcore_barrier()
    pltpu.sync_copy(
        acc.at[pl.ds(sid * RPT, RPT)],
        out_hbm.at[cid, pl.ds(sid * RPT, RPT)],
    )


BLK = 256
GRID = NP // BLK


def _k2a_body(x_ref, w1_ref, ha_ref, hb_ref):
    h = jnp.dot(x_ref[...], w1_ref[...], preferred_element_type=jnp.float32)
    ha_ref[...] = h[:, :128]
    hb_ref[...] = h[:, 128:]


_k2a = pl.pallas_call(
    _k2a_body,
    grid=(GRID,),
    in_specs=[
        pl.BlockSpec((BLK, IN_DIM), lambda i: (i, 0)),
        pl.BlockSpec((IN_DIM, HID), lambda i: (0, 0)),
    ],
    out_specs=[
        pl.BlockSpec((BLK, 128), lambda i: (i, 0)),
        pl.BlockSpec((BLK, 128), lambda i: (i, 0)),
    ],
    out_shape=[
        jax.ShapeDtypeStruct((NP, 128), jnp.float32),
        jax.ShapeDtypeStruct((NP, 128), jnp.float32),
    ],
)


def _k2b_body(deg0_ref, deg1_ref, hra_ref, hrb_ref, dinv_ref, ha_ref, hb_ref):
    deg = deg0_ref[:, 0] + deg1_ref[:, 0] + 1.0
    dinv = lax.rsqrt(deg)[:, None]
    dinv_ref[...] = jnp.broadcast_to(dinv, (BLK, 128))
    ha_ref[...] = hra_ref[...] * dinv
    hb_ref[...] = hrb_ref[...] * dinv


_k2b = pl.pallas_call(
    _k2b_body,
    grid=(GRID,),
    in_specs=[
        pl.BlockSpec((BLK, 128), lambda i: (i, 0)),
        pl.BlockSpec((BLK, 128), lambda i: (i, 0)),
        pl.BlockSpec((BLK, 128), lambda i: (i, 0)),
        pl.BlockSpec((BLK, 128), lambda i: (i, 0)),
    ],
    out_specs=[
        pl.BlockSpec((BLK, 128), lambda i: (i, 0)),
        pl.BlockSpec((BLK, 128), lambda i: (i, 0)),
        pl.BlockSpec((BLK, 128), lambda i: (i, 0)),
    ],
    out_shape=[
        jax.ShapeDtypeStruct((NP, 128), jnp.float32),
        jax.ShapeDtypeStruct((NP, 128), jnp.float32),
        jax.ShapeDtypeStruct((NP, 128), jnp.float32),
    ],
)


def _k4_body(dinv_ref, a1a_ref, a1b_ref, ha_ref, hb_ref, b1_ref, w2_ref,
             h2p_ref):
    dv = dinv_ref[...]
    h1a = jnp.maximum(dv * (a1a_ref[...] + ha_ref[...]) + b1_ref[0, :128], 0.0)
    h1b = jnp.maximum(dv * (a1b_ref[...] + hb_ref[...]) + b1_ref[0, 128:], 0.0)
    h2 = jnp.dot(h1a, w2_ref[:128], preferred_element_type=jnp.float32)
    h2 = h2 + jnp.dot(h1b, w2_ref[128:], preferred_element_type=jnp.float32)
    h2p_ref[...] = jnp.concatenate(
        [h2 * dv[:, :OUT], jnp.zeros((BLK, 128 - OUT), jnp.float32)], axis=1
    )


_k4 = pl.pallas_call(
    _k4_body,
    grid=(GRID,),
    in_specs=[
        pl.BlockSpec((BLK, 128), lambda i: (i, 0)),
        pl.BlockSpec((BLK, 128), lambda i: (i, 0)),
        pl.BlockSpec((BLK, 128), lambda i: (i, 0)),
        pl.BlockSpec((BLK, 128), lambda i: (i, 0)),
        pl.BlockSpec((BLK, 128), lambda i: (i, 0)),
        pl.BlockSpec((1, HID), lambda i: (0, 0)),
        pl.BlockSpec((HID, OUT), lambda i: (0, 0)),
    ],
    out_specs=pl.BlockSpec((BLK, 128), lambda i: (i, 0)),
    out_shape=jax.ShapeDtypeStruct((NP, 128), jnp.float32),
)


def _k6_body(dinv_ref, p0_ref, p1_ref, h2p_ref, b2_ref, out_ref):
    out_ref[...] = (
        dinv_ref[:, :OUT]
        * (p0_ref[:, :OUT] + p1_ref[:, :OUT] + h2p_ref[:, :OUT])
        + b2_ref[0, :]
    )


_k6 = pl.pallas_call(
    _k6_body,
    grid=(GRID,),
    in_specs=[
        pl.BlockSpec((BLK, 128), lambda i: (i, 0)),
        pl.BlockSpec((BLK, 128), lambda i: (i, 0)),
        pl.BlockSpec((BLK, 128), lambda i: (i, 0)),
        pl.BlockSpec((BLK, 128), lambda i: (i, 0)),
        pl.BlockSpec((1, OUT), lambda i: (0, 0)),
    ],
    out_specs=pl.BlockSpec((BLK, OUT), lambda i: (i, 0)),
    out_shape=jax.ShapeDtypeStruct((NP, OUT), jnp.float32),
)


@jax.jit
def kernel(x, edge_index, W1, b1, W2, b2):
    ei = edge_index.astype(jnp.int32)
    pad = N + jnp.arange(EP - E, dtype=jnp.int32) % (NP - N)
    src = jnp.concatenate([ei[0], pad]).reshape(EP // CH, CH)
    dst = jnp.concatenate([ei[1], pad]).reshape(EP // CH, CH)
    xp = jnp.zeros((NP, IN_DIM), jnp.float32).at[:N].set(x)
    zeros128 = jnp.zeros((NP, 128), jnp.float32)
    ones128 = jnp.ones((CH, 128), jnp.float32)

    hra, hrb = _k2a(xp, W1)
    degp = _k1_deg(dst, zeros128, ones128)
    dinv, ha, hb = _k2b(degp[0], degp[1], hra, hrb)
    a1 = _k3_agg1(src, dst, ha, hb, zeros128)
    h2p = _k4(dinv, a1[0], a1[1], ha, hb, b1.reshape(1, HID), W2)
    a2 = _k5_agg2(src, dst, h2p, zeros128)
    out = _k6(dinv, a2[0], a2[1], h2p, b2.reshape(1, OUT))
    return out[:N]


# speedup vs baseline: 14.2149x; 1.0073x over previous
import functools

import jax
import jax.numpy as jnp
from jax import lax
from jax.experimental import pallas as pl
from jax.experimental.pallas import tpu as pltpu
from jax.experimental.pallas import tpu_sc as plsc

N = 10000
E = 160000
IN_DIM = 256
HID = 256
OUT = 64

NP = 10240
EP = 163840
CH = 64
NTILES = 16
RPT = NP // NTILES

_mesh = plsc.VectorSubcoreMesh(core_axis_name="c", subcore_axis_name="s")


C1 = 128


@functools.partial(
    pl.kernel,
    out_type=jax.ShapeDtypeStruct((2, NP, 128), jnp.float32),
    mesh=_mesh,
    scratch_types=[
        pltpu.VMEM((EP // 32 // C1, C1), jnp.int32),
        pltpu.VMEM((C1, 128), jnp.float32),
        pltpu.VMEM_SHARED((NP, 128), jnp.float32),
        pltpu.SemaphoreType.DMA,
    ],
)
def _k1_deg(dst_hbm, zeros_hbm, ones_hbm, out_hbm, dst_all, ones_v, acc, sem):
    cid = lax.axis_index("c")
    sid = lax.axis_index("s")
    nch = EP // 32 // C1
    pltpu.sync_copy(
        zeros_hbm.at[pl.ds(sid * RPT, RPT)],
        acc.at[pl.ds(sid * RPT, RPT)],
    )
    wid = cid * NTILES + sid
    pltpu.sync_copy(dst_hbm.at[pl.ds(wid * nch, nch)], dst_all)
    pltpu.sync_copy(ones_hbm, ones_v)
    plsc.subcore_barrier()

    def fire(c, carry):
        pltpu.async_copy(ones_v, acc.at[dst_all.at[c]], sem, add=True)
        return carry

    lax.fori_loop(0, nch, fire, 0)

    def drain(c, carry):
        pltpu.make_async_copy(ones_hbm, ones_v, sem).wait()
        return carry

    lax.fori_loop(0, nch, drain, 0)
    plsc.subcore_barrier()
    pltpu.sync_copy(
        acc.at[pl.ds(sid * RPT, RPT)],
        out_hbm.at[cid, pl.ds(sid * RPT, RPT)],
    )


NBUF = 4
SEC = 40


def _agg_pipeline(tbl, acc, src_hbm, dst_hbm, chunk0, nch,
                  src_all, dst_all, rows, gsems, ssems):
    def wait_gather(b):
        pltpu.make_async_copy(tbl.at[src_all.at[0]], rows.at[b], gsems[b]).wait()

    def wait_scatter(b):
        pltpu.make_async_copy(tbl.at[src_all.at[0]], rows.at[b], ssems[b]).wait()

    for sec in range(nch // SEC):
        pltpu.sync_copy(src_hbm.at[pl.ds(chunk0 + sec * SEC, SEC)], src_all)
        pltpu.sync_copy(dst_hbm.at[pl.ds(chunk0 + sec * SEC, SEC)], dst_all)
        for b in range(NBUF):
            pltpu.async_copy(tbl.at[src_all.at[b]], rows.at[b], gsems[b])

        def body(g, carry):
            for b in range(NBUF):
                wait_gather(b)
                pltpu.async_copy(
                    rows.at[b], acc.at[dst_all.at[g * NBUF + b]], ssems[b],
                    add=True,
                )
            for b in range(NBUF):
                wait_scatter(b)
                nxt = g * NBUF + b + NBUF

                @pl.when(nxt < SEC)
                def _():
                    pltpu.async_copy(
                        tbl.at[src_all.at[nxt]], rows.at[b], gsems[b]
                    )
            return carry

        lax.fori_loop(0, SEC // NBUF, body, 0)


@functools.partial(
    pl.kernel,
    out_type=jax.ShapeDtypeStruct((2, NP, 128), jnp.float32),
    mesh=_mesh,
    scratch_types=[
        pltpu.VMEM((SEC, CH), jnp.int32),
        pltpu.VMEM((SEC, CH), jnp.int32),
        pltpu.VMEM((NBUF, CH, 128), jnp.float32),
        pltpu.VMEM_SHARED((NP, 128), jnp.float32),
    ]
    + [pltpu.SemaphoreType.DMA] * (2 * NBUF),
)
def _k3_agg1(src_hbm, dst_hbm, ha_hbm, hb_hbm, zeros_hbm, out_hbm,
             src_all, dst_all, rows, acc, *sems):
    gsems, ssems = sems[:NBUF], sems[NBUF:]
    cid = lax.axis_index("c")
    sid = lax.axis_index("s")
    nch = EP // NTILES // CH
    pltpu.sync_copy(
        zeros_hbm.at[pl.ds(sid * RPT, RPT)], acc.at[pl.ds(sid * RPT, RPT)]
    )
    plsc.subcore_barrier()

    @pl.when(cid == 0)
    def _():
        _agg_pipeline(ha_hbm, acc, src_hbm, dst_hbm, sid * nch, nch,
                      src_all, dst_all, rows, gsems, ssems)

    @pl.when(cid == 1)
    def _():
        _agg_pipeline(hb_hbm, acc, src_hbm, dst_hbm, sid * nch, nch,
                      src_all, dst_all, rows, gsems, ssems)

    plsc.subcore_barrier()
    pltpu.sync_copy(
        acc.at[pl.ds(sid * RPT, RPT)],
        out_hbm.at[cid, pl.ds(sid * RPT, RPT)],
    )


@functools.partial(
    pl.kernel,
    out_type=jax.ShapeDtypeStruct((2, NP, 128), jnp.float32),
    mesh=_mesh,
    scratch_types=[
        pltpu.VMEM((SEC, CH), jnp.int32),
        pltpu.VMEM((SEC, CH), jnp.int32),
        pltpu.VMEM((NBUF, CH, 128), jnp.float32),
        pltpu.VMEM_SHARED((NP, 128), jnp.float32),
    ]
    + [pltpu.SemaphoreType.DMA] * (2 * NBUF),
)
def _k5_agg2(src_hbm, dst_hbm, h2p_hbm, zeros_hbm, out_hbm,
             src_all, dst_all, rows, acc, *sems):
    gsems, ssems = sems[:NBUF], sems[NBUF:]
    cid = lax.axis_index("c")
    sid = lax.axis_index("s")
    nch = EP // 32 // CH
    pltpu.sync_copy(
        zeros_hbm.at[pl.ds(sid * RPT, RPT)],
        acc.at[pl.ds(sid * RPT, RPT)],
    )
    wid = cid * NTILES + sid
    plsc.subcore_barrier()
    _agg_pipeline(h2p_hbm, acc, src_hbm, dst_hbm, wid * nch, nch,
                  src_all, dst_all, rows, gsems, ssems)
    plsc.subcore_barrier()
    pltpu.sync_copy(
        acc.at[pl.ds(sid * RPT, RPT)],
        out_hbm.at[cid, pl.ds(sid * RPT, RPT)],
    )


BLK = 256
GRID = NP // BLK


def _k2a_body(x_ref, w1_ref, ha_ref, hb_ref):
    h = jnp.dot(x_ref[...], w1_ref[...], preferred_element_type=jnp.float32)
    ha_ref[...] = h[:, :128]
    hb_ref[...] = h[:, 128:]


_k2a = pl.pallas_call(
    _k2a_body,
    grid=(GRID,),
    in_specs=[
        pl.BlockSpec((BLK, IN_DIM), lambda i: (i, 0)),
        pl.BlockSpec((IN_DIM, HID), lambda i: (0, 0)),
    ],
    out_specs=[
        pl.BlockSpec((BLK, 128), lambda i: (i, 0)),
        pl.BlockSpec((BLK, 128), lambda i: (i, 0)),
    ],
    out_shape=[
        jax.ShapeDtypeStruct((NP, 128), jnp.float32),
        jax.ShapeDtypeStruct((NP, 128), jnp.float32),
    ],
)


def _k2b_body(deg0_ref, deg1_ref, hra_ref, hrb_ref, dinv_ref, ha_ref, hb_ref):
    deg = deg0_ref[:, 0] + deg1_ref[:, 0] + 1.0
    dinv = lax.rsqrt(deg)[:, None]
    dinv_ref[...] = jnp.broadcast_to(dinv, (BLK, 128))
    ha_ref[...] = hra_ref[...] * dinv
    hb_ref[...] = hrb_ref[...] * dinv


_k2b = pl.pallas_call(
    _k2b_body,
    grid=(GRID,),
    in_specs=[
        pl.BlockSpec((BLK, 128), lambda i: (i, 0)),
        pl.BlockSpec((BLK, 128), lambda i: (i, 0)),
        pl.BlockSpec((BLK, 128), lambda i: (i, 0)),
        pl.BlockSpec((BLK, 128), lambda i: (i, 0)),
    ],
    out_specs=[
        pl.BlockSpec((BLK, 128), lambda i: (i, 0)),
        pl.BlockSpec((BLK, 128), lambda i: (i, 0)),
        pl.BlockSpec((BLK, 128), lambda i: (i, 0)),
    ],
    out_shape=[
        jax.ShapeDtypeStruct((NP, 128), jnp.float32),
        jax.ShapeDtypeStruct((NP, 128), jnp.float32),
        jax.ShapeDtypeStruct((NP, 128), jnp.float32),
    ],
)


def _k4_body(dinv_ref, a1a_ref, a1b_ref, ha_ref, hb_ref, b1_ref, w2_ref,
             h2p_ref):
    dv = dinv_ref[...]
    h1a = jnp.maximum(dv * (a1a_ref[...] + ha_ref[...]) + b1_ref[0, :128], 0.0)
    h1b = jnp.maximum(dv * (a1b_ref[...] + hb_ref[...]) + b1_ref[0, 128:], 0.0)
    h2 = jnp.dot(h1a, w2_ref[:128], preferred_element_type=jnp.float32)
    h2 = h2 + jnp.dot(h1b, w2_ref[128:], preferred_element_type=jnp.float32)
    h2p_ref[...] = jnp.concatenate(
        [h2 * dv[:, :OUT], jnp.zeros((BLK, 128 - OUT), jnp.float32)], axis=1
    )


_k4 = pl.pallas_call(
    _k4_body,
    grid=(GRID,),
    in_specs=[
        pl.BlockSpec((BLK, 128), lambda i: (i, 0)),
        pl.BlockSpec((BLK, 128), lambda i: (i, 0)),
        pl.BlockSpec((BLK, 128), lambda i: (i, 0)),
        pl.BlockSpec((BLK, 128), lambda i: (i, 0)),
        pl.BlockSpec((BLK, 128), lambda i: (i, 0)),
        pl.BlockSpec((1, HID), lambda i: (0, 0)),
        pl.BlockSpec((HID, OUT), lambda i: (0, 0)),
    ],
    out_specs=pl.BlockSpec((BLK, 128), lambda i: (i, 0)),
    out_shape=jax.ShapeDtypeStruct((NP, 128), jnp.float32),
)


def _k6_body(dinv_ref, p0_ref, p1_ref, h2p_ref, b2_ref, out_ref):
    out_ref[...] = (
        dinv_ref[:, :OUT]
        * (p0_ref[:, :OUT] + p1_ref[:, :OUT] + h2p_ref[:, :OUT])
        + b2_ref[0, :]
    )


_k6 = pl.pallas_call(
    _k6_body,
    grid=(GRID,),
    in_specs=[
        pl.BlockSpec((BLK, 128), lambda i: (i, 0)),
        pl.BlockSpec((BLK, 128), lambda i: (i, 0)),
        pl.BlockSpec((BLK, 128), lambda i: (i, 0)),
        pl.BlockSpec((BLK, 128), lambda i: (i, 0)),
        pl.BlockSpec((1, OUT), lambda i: (0, 0)),
    ],
    out_specs=pl.BlockSpec((BLK, OUT), lambda i: (i, 0)),
    out_shape=jax.ShapeDtypeStruct((NP, OUT), jnp.float32),
)


@jax.jit
def kernel(x, edge_index, W1, b1, W2, b2):
    ei = edge_index.astype(jnp.int32)
    pad = N + jnp.arange(EP - E, dtype=jnp.int32) % (NP - N)
    src = jnp.concatenate([ei[0], pad]).reshape(EP // CH, CH)
    dst = jnp.concatenate([ei[1], pad]).reshape(EP // CH, CH)
    xp = jnp.zeros((NP, IN_DIM), jnp.float32).at[:N].set(x)
    zeros128 = jnp.zeros((NP, 128), jnp.float32)
    ones128 = jnp.ones((C1, 128), jnp.float32)

    hra, hrb = _k2a(xp, W1)
    degp = _k1_deg(dst.reshape(EP // C1, C1), zeros128, ones128)
    dinv, ha, hb = _k2b(degp[0], degp[1], hra, hrb)
    a1 = _k3_agg1(src, dst, ha, hb, zeros128)
    h2p = _k4(dinv, a1[0], a1[1], ha, hb, b1.reshape(1, HID), W2)
    a2 = _k5_agg2(src, dst, h2p, zeros128)
    out = _k6(dinv, a2[0], a2[1], h2p, b2.reshape(1, OUT))
    return out[:N]
